# baseline (speedup 1.0000x reference)
import jax
import jax.numpy as jnp
from jax.experimental import pallas as pl
from jax.experimental.pallas import tpu as pltpu

_H = 56
_W = 56
_WP = 64
_C = 128
_F = 512
_EPS = 1e-6
_STRIP = 8
_NSTRIP = _H // _STRIP


def _body(x_ref, taps_ref, dwb_ref, lng_ref, lnb_ref, w1_ref, b1_ref,
          w2_ref, b2_ref, gamma_ref, o_ref, pad_ref, shb_ref, yc_ref):
    pad_ref[0:3, :, :] = jnp.zeros((3, _WP + 8, _C), jnp.float32)
    pad_ref[3 + _H:, :, :] = jnp.zeros((3, _WP + 8, _C), jnp.float32)
    pad_ref[3:3 + _H, 0:3, :] = jnp.zeros((_H, 3, _C), jnp.float32)
    pad_ref[3:3 + _H, 3 + _W:6 + _W, :] = jnp.zeros((_H, 3, _C), jnp.float32)
    pad_ref[3:3 + _H, 3:3 + _W, :] = x_ref[0]
    for j in range(7):
        shb_ref[j] = pad_ref[:, j:j + _WP, :].astype(jnp.bfloat16)

    def conv_strip(h0):
        acc = [jnp.zeros((_WP, _C), jnp.bfloat16) for _ in range(_STRIP)]
        for j in range(7):
            for k in range(_STRIP + 6):
                v = shb_ref[j, h0 + k]
                for i in range(7):
                    r = k - i
                    if 0 <= r < _STRIP:
                        acc[r] = acc[r] + v * taps_ref[7 * i + j]
        return jnp.stack(acc)

    def mlp_strip(acc, h0):
        y = acc[:, :_W, :].astype(jnp.float32) + dwb_ref[...]
        mu = jnp.mean(y, axis=-1, keepdims=True)
        d = y - mu
        var = jnp.mean(d * d, axis=-1, keepdims=True)
        yn = d * jax.lax.rsqrt(var + _EPS) * lng_ref[...] + lnb_ref[...]
        yn2 = yn.reshape(_STRIP * _W, _C).astype(jnp.bfloat16)
        h1 = jnp.dot(yn2, w1_ref[...], preferred_element_type=jnp.float32)
        h1 = h1 + b1_ref[...]
        h1 = h1 + h1 * jax.lax.erf(h1 * 0.7071067811865476)
        y2 = jnp.dot(h1.astype(jnp.bfloat16), w2_ref[...],
                     preferred_element_type=jnp.float32)
        y2 = (y2 + b2_ref[...]) * gamma_ref[...]
        o_ref[0, pl.ds(h0, _STRIP)] = (
            x_ref[0, pl.ds(h0, _STRIP)] + y2.reshape(_STRIP, _W, _C))

    def piped(t, carry):
        acc_next = conv_strip((t + 1) * _STRIP)
        mlp_strip(carry, t * _STRIP)
        return acc_next

    last = jax.lax.fori_loop(0, _NSTRIP - 1, piped, conv_strip(0),
                             unroll=False)
    mlp_strip(last, (_NSTRIP - 1) * _STRIP)


@jax.jit
def kernel(x, dw_w, dw_b, ln_g, ln_b, w1, b1, w2, b2, gamma):
    n = x.shape[0]
    xt = jnp.transpose(x, (0, 2, 3, 1))
    taps = jnp.transpose(dw_w[:, 0, :, :], (1, 2, 0)).reshape(49, _C)
    taps = taps.astype(jnp.bfloat16)
    row = lambda v: v.reshape(1, -1)
    out_nhwc = pl.pallas_call(
        _body,
        grid=(n,),
        in_specs=[
            pl.BlockSpec((1, _H, _W, _C), lambda b: (b, 0, 0, 0)),
            pl.BlockSpec((49, _C), lambda b: (0, 0)),
            pl.BlockSpec((1, _C), lambda b: (0, 0)),
            pl.BlockSpec((1, _C), lambda b: (0, 0)),
            pl.BlockSpec((1, _C), lambda b: (0, 0)),
            pl.BlockSpec((_C, _F), lambda b: (0, 0)),
            pl.BlockSpec((1, _F), lambda b: (0, 0)),
            pl.BlockSpec((_F, _C), lambda b: (0, 0)),
            pl.BlockSpec((1, _C), lambda b: (0, 0)),
            pl.BlockSpec((1, _C), lambda b: (0, 0)),
        ],
        out_specs=pl.BlockSpec((1, _H, _W, _C), lambda b: (b, 0, 0, 0)),
        out_shape=jax.ShapeDtypeStruct((n, _H, _W, _C), jnp.float32),
        scratch_shapes=[
            pltpu.VMEM((_H + 6, _WP + 8, _C), jnp.float32),
            pltpu.VMEM((7, _H + 6, _WP, _C), jnp.bfloat16),
            pltpu.VMEM((2, _STRIP, _WP, _C), jnp.bfloat16),
        ],
        compiler_params=pltpu.CompilerParams(
            dimension_semantics=("parallel",),
            vmem_limit_bytes=48 * 1024 * 1024,
            flags={"XLA_TPU_STORE_TO_LOAD_FORWARDING_WINDOW": 12288},
        ),
        name="convnext_block",
    )(xt, taps, row(dw_b), row(ln_g), row(ln_b), w1.astype(jnp.bfloat16),
      row(b1), (0.5 * w2).astype(jnp.bfloat16), row(b2), row(gamma))
    return jnp.transpose(out_nhwc, (0, 3, 1, 2))


# speedup vs baseline: 1.0031x; 1.0031x over previous
import jax
import jax.numpy as jnp
from jax.experimental import pallas as pl
from jax.experimental.pallas import tpu as pltpu

_H = 56
_W = 56
_WP = 64
_C = 128
_F = 512
_EPS = 1e-6
_STRIP = 8
_NSTRIP = _H // _STRIP


def _body(x_ref, taps_ref, dwb_ref, lng_ref, lnb_ref, w1_ref, b1_ref,
          w2_ref, b2_ref, gamma_ref, o_ref, pad_ref, shb_ref):
    pad_ref[0:3, :, :] = jnp.zeros((3, _WP + 8, _C), jnp.float32)
    pad_ref[3 + _H:, :, :] = jnp.zeros((3, _WP + 8, _C), jnp.float32)
    pad_ref[3:3 + _H, 0:3, :] = jnp.zeros((_H, 3, _C), jnp.float32)
    pad_ref[3:3 + _H, 3 + _W:6 + _W, :] = jnp.zeros((_H, 3, _C), jnp.float32)
    pad_ref[3:3 + _H, 3:3 + _W, :] = x_ref[0]
    for j in range(7):
        shb_ref[j] = pad_ref[:, j:j + _WP, :].astype(jnp.bfloat16)

    def conv_strip(h0):
        acc = [jnp.zeros((_WP, _C), jnp.bfloat16) for _ in range(_STRIP)]
        for j in range(7):
            for k in range(_STRIP + 6):
                v = shb_ref[j, h0 + k]
                for i in range(7):
                    r = k - i
                    if 0 <= r < _STRIP:
                        acc[r] = acc[r] + v * taps_ref[7 * i + j]
        return jnp.stack(acc)

    def mlp_strip(acc, h0):
        y = acc[:, :_W, :].astype(jnp.float32) + dwb_ref[...]
        mu = jnp.mean(y, axis=-1, keepdims=True)
        d = y - mu
        var = jnp.mean(d * d, axis=-1, keepdims=True)
        yn = d * jax.lax.rsqrt(var + _EPS) * lng_ref[...] + lnb_ref[...]
        yn2 = yn.reshape(_STRIP * _W, _C).astype(jnp.bfloat16)
        h1 = jnp.dot(yn2, w1_ref[...], preferred_element_type=jnp.float32)
        h1 = h1 + b1_ref[...]
        h1 = h1 + h1 * jax.lax.erf(h1 * 0.7071067811865476)
        y2 = jnp.dot(h1.astype(jnp.bfloat16), w2_ref[...],
                     preferred_element_type=jnp.float32)
        y2 = (y2 + b2_ref[...]) * gamma_ref[...]
        o_ref[0, pl.ds(h0, _STRIP)] = (
            x_ref[0, pl.ds(h0, _STRIP)] + y2.reshape(_STRIP, _W, _C))

    def piped(t, carry):
        acc_next = conv_strip((t + 1) * _STRIP)
        mlp_strip(carry, t * _STRIP)
        return acc_next

    last = jax.lax.fori_loop(0, _NSTRIP - 1, piped, conv_strip(0),
                             unroll=False)
    mlp_strip(last, (_NSTRIP - 1) * _STRIP)


@jax.jit
def kernel(x, dw_w, dw_b, ln_g, ln_b, w1, b1, w2, b2, gamma):
    n = x.shape[0]
    xt = jnp.transpose(x, (0, 2, 3, 1))
    taps = jnp.transpose(dw_w[:, 0, :, :], (1, 2, 0)).reshape(49, _C)
    taps = taps.astype(jnp.bfloat16)
    row = lambda v: v.reshape(1, -1)
    out_nhwc = pl.pallas_call(
        _body,
        grid=(n,),
        in_specs=[
            pl.BlockSpec((1, _H, _W, _C), lambda b: (b, 0, 0, 0)),
            pl.BlockSpec((49, _C), lambda b: (0, 0)),
            pl.BlockSpec((1, _C), lambda b: (0, 0)),
            pl.BlockSpec((1, _C), lambda b: (0, 0)),
            pl.BlockSpec((1, _C), lambda b: (0, 0)),
            pl.BlockSpec((_C, _F), lambda b: (0, 0)),
            pl.BlockSpec((1, _F), lambda b: (0, 0)),
            pl.BlockSpec((_F, _C), lambda b: (0, 0)),
            pl.BlockSpec((1, _C), lambda b: (0, 0)),
            pl.BlockSpec((1, _C), lambda b: (0, 0)),
        ],
        out_specs=pl.BlockSpec((1, _H, _W, _C), lambda b: (b, 0, 0, 0)),
        out_shape=jax.ShapeDtypeStruct((n, _H, _W, _C), jnp.float32),
        scratch_shapes=[
            pltpu.VMEM((_H + 6, _WP + 8, _C), jnp.float32),
            pltpu.VMEM((7, _H + 6, _WP, _C), jnp.bfloat16),
        ],
        compiler_params=pltpu.CompilerParams(
            dimension_semantics=("parallel",),
            vmem_limit_bytes=48 * 1024 * 1024,
        ),
        name="convnext_block",
    )(xt, taps, row(dw_b), row(ln_g), row(ln_b), w1.astype(jnp.bfloat16),
      row(b1), (0.5 * w2).astype(jnp.bfloat16), row(b2), row(gamma))
    return jnp.transpose(out_nhwc, (0, 3, 1, 2))
